# Initial kernel scaffold; baseline (speedup 1.0000x reference)
import jax
import jax.numpy as jnp
from jax.experimental import pallas as pl


def kernel(x, edge_index, W1, a1s, a1d, b1, W2, a2s, a2d, b2):
    raise NotImplementedError("write your pallas kernel here")


# speedup vs baseline: 21.1270x; 21.1270x over previous
import functools

import jax
import jax.numpy as jnp
from jax import lax
from jax.experimental import pallas as pl
from jax.experimental.pallas import tpu as pltpu
from jax.experimental.pallas import tpu_sc as plsc

NN = 10000
EE = 320000
E2 = EE + NN
NP = 10240
NCORE = 2
NSUB = 16
CH = 128
GG = 8
TCH1 = 168
TCH2 = 88
RPT = NP // NSUB
BM = 2000


def _gat_pre_l1(x_ref, w_ref, as_ref, ad_ref, hs_ref, als_ref, ald_ref):
    h = jnp.dot(x_ref[...], w_ref[...], preferred_element_type=jnp.float32)
    hs_ref[0] = h[:, :128]
    hs_ref[1] = h[:, 128:]
    als_ref[...] = h @ as_ref[...]
    ald_ref[...] = h @ ad_ref[...]


def _gat_pre_l2(y_ref, b_ref, w_ref, as_ref, ad_ref, h_ref, als_ref, ald_ref):
    ya = jnp.maximum(y_ref[0] + b_ref[0, :128], 0.0)
    yb = jnp.maximum(y_ref[1] + b_ref[0, 128:], 0.0)
    h = (jnp.dot(ya, w_ref[:128, :], preferred_element_type=jnp.float32)
         + jnp.dot(yb, w_ref[128:, :], preferred_element_type=jnp.float32))
    h_ref[...] = h
    als_ref[...] = h @ as_ref[...]
    ald_ref[...] = h @ ad_ref[...]


def _final_body(acc_ref, den_ref, b_ref, o_ref):
    den = den_ref[0] + den_ref[1]
    row = (acc_ref[0] + acc_ref[1]) / den + b_ref[...]
    m = jnp.max(row, axis=-1, keepdims=True)
    ex = jnp.exp(row - m)
    o_ref[...] = row - m - jnp.log(jnp.sum(ex, axis=-1, keepdims=True))


def _make_sc_gat(feature_split):
    mesh = plsc.VectorSubcoreMesh(core_axis_name="c", subcore_axis_name="s")
    TCH = TCH1 if feature_split else TCH2
    NG = TCH // GG
    EPT = TCH * CH
    if feature_split:
        out_type = jax.ShapeDtypeStruct((NCORE, NP, 128), jnp.float32)
    else:
        out_type = [jax.ShapeDtypeStruct((NCORE, NP, 128), jnp.float32),
                    jax.ShapeDtypeStruct((NCORE, NP), jnp.float32)]

    @functools.partial(
        pl.kernel,
        mesh=mesh,
        out_type=out_type,
        compiler_params=pltpu.CompilerParams(needs_layout_passes=False),
        scratch_types=[
            pltpu.VMEM((NN,), jnp.float32),
            pltpu.VMEM((NN,), jnp.float32),
            pltpu.VMEM((GG, CH), jnp.int32),
            pltpu.VMEM((GG, CH), jnp.int32),
            pltpu.VMEM((GG, CH), jnp.float32),
            pltpu.VMEM((CH, 128), jnp.float32),
            pltpu.VMEM((RPT,), jnp.float32),
            pltpu.VMEM_SHARED((NP, 128), jnp.float32),
            pltpu.VMEM_SHARED((NP,), jnp.float32),
            pltpu.SemaphoreType.DMA,
        ],
    )
    def sc_gat(hs_hbm, als_hbm, ald_hbm, src_hbm, dst_hbm, *refs):
        if feature_split:
            (out_hbm, als_v, ald_v, src_v, dst_v, ex_v, rows_v, den_v,
             acc_sh, den_sh, sem) = refs
        else:
            (out_hbm, dout_hbm, als_v, ald_v, src_v, dst_v, ex_v, rows_v,
             den_v, acc_sh, den_sh, sem) = refs
        c = lax.axis_index("c")
        s = lax.axis_index("s")
        zero16 = jnp.zeros((16,), jnp.float32)

        def _zrow(r, _):
            for f in range(8):
                rows_v[r, pl.ds(f * 16, 16)] = zero16
            return 0
        lax.fori_loop(0, CH, _zrow, 0)

        def _zden(r, _):
            den_v[pl.ds(r * 16, 16)] = zero16
            return 0
        lax.fori_loop(0, RPT // 16, _zden, 0)

        base_row = s * RPT
        for kk in range(RPT // CH):
            pltpu.sync_copy(rows_v, acc_sh.at[pl.ds(base_row + kk * CH, CH)])
        pltpu.sync_copy(den_v, den_sh.at[pl.ds(base_row, RPT)])

        pltpu.sync_copy(als_hbm, als_v)
        pltpu.sync_copy(ald_hbm, ald_v)

        plsc.subcore_barrier()

        if feature_split:
            ebase = s * EPT
            coff = c * NN
        else:
            ebase = (c * NSUB + s) * EPT
            coff = 0
        lane = jnp.arange(16, dtype=jnp.int32)

        def _group(g, _):
            if feature_split:
                pltpu.sync_copy(src_hbm.at[s, pl.ds(g * GG, GG)], src_v)
                pltpu.sync_copy(dst_hbm.at[s, pl.ds(g * GG, GG)], dst_v)
            else:
                pltpu.sync_copy(src_hbm.at[c, s, pl.ds(g * GG, GG)], src_v)
                pltpu.sync_copy(dst_hbm.at[c, s, pl.ds(g * GG, GG)], dst_v)

            def _chunk(jj, _):
                for k in range(CH // 16):
                    sl = pl.ds(k * 16, 16)
                    s16 = src_v[jj, sl]
                    d16 = dst_v[jj, sl]
                    av = plsc.load_gather(als_v, [s16])
                    dv = plsc.load_gather(ald_v, [d16])
                    z = av + dv
                    z = jnp.maximum(z, 0.2 * z)
                    exv = jnp.exp(z)
                    eid = ebase + (g * GG + jj) * CH + k * 16 + lane
                    exv = jnp.where(eid < E2, exv, 0.0)
                    ex_v[jj, sl] = exv
                    if feature_split:
                        src_v[jj, sl] = s16 + coff
                pltpu.sync_copy(ex_v.at[jj], den_sh.at[dst_v.at[jj]], add=True)
                pltpu.async_copy(hs_hbm.at[src_v.at[jj]], rows_v, sem).wait()

                def _scale(i, _):
                    exi = plsc.load_gather(
                        ex_v, [jnp.full((16,), jj, jnp.int32),
                               jnp.full((16,), i, jnp.int32)])
                    for f in range(8):
                        fl = pl.ds(f * 16, 16)
                        rows_v[i, fl] = rows_v[i, fl] * exi
                    return 0
                lax.fori_loop(0, CH, _scale, 0)
                pltpu.sync_copy(rows_v, acc_sh.at[dst_v.at[jj]], add=True)
                return 0
            lax.fori_loop(0, GG, _chunk, 0)
            return 0
        lax.fori_loop(0, NG, _group, 0)

        plsc.subcore_barrier()

        if feature_split:
            pltpu.sync_copy(den_sh.at[pl.ds(base_row, RPT)], den_v)

            def _recip(r, _):
                sl = pl.ds(r * 16, 16)
                den_v[sl] = 1.0 / den_v[sl]
                return 0
            lax.fori_loop(0, RPT // 16, _recip, 0)

            for kk in range(RPT // CH):
                pltpu.sync_copy(acc_sh.at[pl.ds(base_row + kk * CH, CH)],
                                rows_v)

                def _norm(i, _):
                    ri = plsc.load_gather(
                        den_v, [jnp.full((16,), kk * CH + i, jnp.int32)])
                    for f in range(8):
                        fl = pl.ds(f * 16, 16)
                        rows_v[i, fl] = rows_v[i, fl] * ri
                    return 0
                lax.fori_loop(0, CH, _norm, 0)
                pltpu.sync_copy(rows_v,
                                out_hbm.at[c, pl.ds(base_row + kk * CH, CH)])
        else:
            for kk in range(RPT // CH):
                sl = pl.ds(base_row + kk * CH, CH)
                pltpu.sync_copy(acc_sh.at[sl], out_hbm.at[c, sl])
            pltpu.sync_copy(den_sh.at[pl.ds(base_row, RPT)],
                            dout_hbm.at[c, pl.ds(base_row, RPT)])

    return sc_gat


_sc_gat_l1 = _make_sc_gat(True)
_sc_gat_l2 = _make_sc_gat(False)


def _pad_edges(v, tch, split_cores):
    ncore = NCORE if split_cores else 1
    total = tch * CH * NSUB * ncore
    pad = jnp.arange(total - E2, dtype=jnp.int32) % NN
    flat = jnp.concatenate([v, pad])
    if split_cores:
        return flat.reshape(NCORE, NSUB, tch, CH)
    return flat.reshape(NSUB, tch, CH)


def kernel(x, edge_index, W1, a1s, a1d, b1, W2, a2s, a2d, b2):
    loop = jnp.arange(NN, dtype=jnp.int32)
    src = jnp.concatenate([edge_index[0], loop])
    dst = jnp.concatenate([edge_index[1], loop])
    src1, dst1 = _pad_edges(src, TCH1, False), _pad_edges(dst, TCH1, False)
    src2, dst2 = _pad_edges(src, TCH2, True), _pad_edges(dst, TCH2, True)

    grid = NN // BM

    hs1, als1, ald1 = pl.pallas_call(
        _gat_pre_l1,
        grid=(grid,),
        in_specs=[
            pl.BlockSpec((BM, 128), lambda i: (i, 0)),
            pl.BlockSpec((128, 256), lambda i: (0, 0)),
            pl.BlockSpec((256, 1), lambda i: (0, 0)),
            pl.BlockSpec((256, 1), lambda i: (0, 0)),
        ],
        out_specs=[
            pl.BlockSpec((2, BM, 128), lambda i: (0, i, 0)),
            pl.BlockSpec((BM, 1), lambda i: (i, 0)),
            pl.BlockSpec((BM, 1), lambda i: (i, 0)),
        ],
        out_shape=[
            jax.ShapeDtypeStruct((2, NN, 128), jnp.float32),
            jax.ShapeDtypeStruct((NN, 1), jnp.float32),
            jax.ShapeDtypeStruct((NN, 1), jnp.float32),
        ],
    )(x, W1, a1s.reshape(256, 1), a1d.reshape(256, 1))

    y1 = _sc_gat_l1(hs1.reshape(2 * NN, 128), als1.reshape(NN),
                    ald1.reshape(NN), src1, dst1)

    h2, als2, ald2 = pl.pallas_call(
        _gat_pre_l2,
        grid=(grid,),
        in_specs=[
            pl.BlockSpec((2, BM, 128), lambda i: (0, i, 0)),
            pl.BlockSpec((1, 256), lambda i: (0, 0)),
            pl.BlockSpec((256, 128), lambda i: (0, 0)),
            pl.BlockSpec((128, 1), lambda i: (0, 0)),
            pl.BlockSpec((128, 1), lambda i: (0, 0)),
        ],
        out_specs=[
            pl.BlockSpec((BM, 128), lambda i: (i, 0)),
            pl.BlockSpec((BM, 1), lambda i: (i, 0)),
            pl.BlockSpec((BM, 1), lambda i: (i, 0)),
        ],
        out_shape=[
            jax.ShapeDtypeStruct((NN, 128), jnp.float32),
            jax.ShapeDtypeStruct((NN, 1), jnp.float32),
            jax.ShapeDtypeStruct((NN, 1), jnp.float32),
        ],
    )(y1, b1.reshape(1, 256), W2, a2s.reshape(128, 1), a2d.reshape(128, 1))

    y2, d2 = _sc_gat_l2(h2, als2.reshape(NN), ald2.reshape(NN), src2, dst2)

    out = pl.pallas_call(
        _final_body,
        grid=(grid,),
        in_specs=[
            pl.BlockSpec((2, BM, 128), lambda i: (0, i, 0)),
            pl.BlockSpec((2, BM, 1), lambda i: (0, i, 0)),
            pl.BlockSpec((1, 128), lambda i: (0, 0)),
        ],
        out_specs=pl.BlockSpec((BM, 128), lambda i: (i, 0)),
        out_shape=jax.ShapeDtypeStruct((NN, 128), jnp.float32),
    )(y2, d2.reshape(2, NP, 1), b2.reshape(1, 128))
    return out
